# Initial kernel scaffold; baseline (speedup 1.0000x reference)
import jax
import jax.numpy as jnp
from jax.experimental import pallas as pl


def kernel(past_exo_cont, past_exo_cat, tables, B, L):
    raise NotImplementedError("write your pallas kernel here")


# speedup vs baseline: 6.7056x; 6.7056x over previous
import functools

import jax
import jax.numpy as jnp
from jax import lax
from jax.experimental import pallas as pl
from jax.experimental.pallas import tpu as pltpu
from jax.experimental.pallas import tpu_sc as plsc

NC, NS = 2, 16
NW = NC * NS
GB = 128


def _make_kernel(N, K, DC, ED, C):
    D_OUT = DC + K * ED
    per_w = N // NW
    n_chunks = per_w // C
    n_sub = C // GB
    mesh = plsc.VectorSubcoreMesh(core_axis_name="c", subcore_axis_name="s")

    @functools.partial(
        pl.kernel,
        out_type=jax.ShapeDtypeStruct((N, D_OUT), jnp.float32),
        mesh=mesh,
        compiler_params=pltpu.CompilerParams(use_tc_tiling_on_sc=False),
        scratch_types=[
            pltpu.VMEM((K, C), jnp.int32),
            pltpu.VMEM((C, DC), jnp.float32),
            pltpu.VMEM((K, C, ED), jnp.float32),
            pltpu.SemaphoreType.DMA,
        ],
    )
    def k(tables_hbm, idx_hbm, cont_hbm, out_hbm, idx_v, cont_v, rows_v, sem):
        wid = lax.axis_index("s") * NC + lax.axis_index("c")
        base0 = wid * per_w

        def chunk_body(ci, _):
            base = base0 + ci * C
            pltpu.sync_copy(idx_hbm.at[:, pl.ds(base, C)], idx_v)
            pltpu.sync_copy(cont_hbm.at[pl.ds(base, C)], cont_v)
            copies = []
            for i in range(K):
                for g in range(n_sub):
                    copies.append(pltpu.async_copy(
                        tables_hbm.at[idx_v.at[i, pl.ds(g * GB, GB)]],
                        rows_v.at[i, pl.ds(g * GB, GB)],
                        sem))
            pltpu.sync_copy(cont_v, out_hbm.at[pl.ds(base, C), pl.ds(0, DC)])
            for cp in copies:
                cp.wait()
            for i in range(K):
                pltpu.sync_copy(
                    rows_v.at[i],
                    out_hbm.at[pl.ds(base, C), pl.ds(DC + i * ED, ED)])
            return ()

        lax.fori_loop(0, n_chunks, chunk_body, (), unroll=False)

    return k


def kernel(past_exo_cont, past_exo_cat, tables, B, L):
    del B, L
    K, VOCAB, ED = tables.shape
    B, L, DC = past_exo_cont.shape
    N = B * L
    idx = past_exo_cat.reshape(N, K).astype(jnp.int32)
    offs = (jnp.arange(K, dtype=jnp.int32) * VOCAB)[:, None]
    idx_t = idx.T + offs
    tables_flat = tables.reshape(K * VOCAB, ED)
    cont_flat = past_exo_cont.reshape(N, DC)
    out = _make_kernel(N, K, DC, ED, 256)(tables_flat, idx_t, cont_flat)
    return out.reshape(B, L, DC + K * ED)


# speedup vs baseline: 7.9613x; 1.1873x over previous
import functools

import jax
import jax.numpy as jnp
from jax import lax
from jax.experimental import pallas as pl
from jax.experimental.pallas import tpu as pltpu
from jax.experimental.pallas import tpu_sc as plsc

NC, NS = 2, 16
NW = NC * NS
GB = 128
LANES = 16


def _make_kernel(N, K, DC, ED, VOCAB, C):
    D_OUT = DC + K * ED
    per_w = N // NW
    n_chunks = per_w // C
    n_sub = C // GB
    assert n_chunks % 2 == 0
    mesh = plsc.VectorSubcoreMesh(core_axis_name="c", subcore_axis_name="s")

    @functools.partial(
        pl.kernel,
        out_type=jax.ShapeDtypeStruct((N, D_OUT), jnp.float32),
        mesh=mesh,
        compiler_params=pltpu.CompilerParams(
            use_tc_tiling_on_sc=False, needs_layout_passes=False),
        scratch_types=[
            pltpu.VMEM((2, C, K), jnp.int32),
            pltpu.VMEM((2, K, C), jnp.int32),
            pltpu.VMEM((2, C, DC), jnp.float32),
            pltpu.VMEM((2, K, C, ED), jnp.float32),
            pltpu.SemaphoreType.DMA,
            pltpu.SemaphoreType.DMA,
            pltpu.SemaphoreType.DMA,
            pltpu.SemaphoreType.DMA,
            pltpu.SemaphoreType.DMA,
        ],
    )
    def k(tables_hbm, idx_hbm, cont_hbm, out_hbm,
          raw_v, idx_v, cont_v, rows_v, gsem, isem0, isem1, osem0, osem1):
        isem = (isem0, isem1)
        osem = (osem0, osem1)
        wid = lax.axis_index("s") * NC + lax.axis_index("c")
        base0 = wid * per_w

        def start_loads(ci, b):
            base = base0 + ci * C
            pltpu.async_copy(idx_hbm.at[pl.ds(base, C)], raw_v.at[b], isem[b])
            pltpu.async_copy(cont_hbm.at[pl.ds(base, C)], cont_v.at[b], isem[b])

        def wait_loads(b):
            pltpu.make_async_copy(
                idx_hbm.at[pl.ds(0, C)], raw_v.at[b], isem[b]).wait()
            pltpu.make_async_copy(
                cont_hbm.at[pl.ds(0, C)], cont_v.at[b], isem[b]).wait()

        def drain_outs(b):
            for i in range(K):
                pltpu.make_async_copy(
                    rows_v.at[b, i],
                    out_hbm.at[pl.ds(0, C), pl.ds(DC + i * ED, ED)],
                    osem[b]).wait()
            pltpu.make_async_copy(
                cont_v.at[b],
                out_hbm.at[pl.ds(0, C), pl.ds(0, DC)], osem[b]).wait()

        row_off = lax.iota(jnp.int32, LANES)

        def compute_idx(b):
            raw_b = raw_v.at[b]
            for g in range(C // LANES):
                rows = row_off + (g * LANES)
                for i in range(K):
                    cols = jnp.full((LANES,), i, jnp.int32)
                    v = plsc.load_gather(raw_b, [rows, cols])
                    idx_v[b, i, pl.ds(g * LANES, LANES)] = v + (i * VOCAB)

        def run_chunk(ci, b, first, last):
            base = base0 + ci * C
            wait_loads(b)
            if not first:
                drain_outs(1 - b)

            if not last:
                start_loads(ci + 1, 1 - b)
            compute_idx(b)
            copies = []
            for i in range(K):
                for g in range(n_sub):
                    copies.append(pltpu.async_copy(
                        tables_hbm.at[idx_v.at[b, i, pl.ds(g * GB, GB)]],
                        rows_v.at[b, i, pl.ds(g * GB, GB)],
                        gsem))
            for cp in copies:
                cp.wait()
            for i in range(K):
                pltpu.async_copy(
                    rows_v.at[b, i],
                    out_hbm.at[pl.ds(base, C), pl.ds(DC + i * ED, ED)],
                    osem[b])
            pltpu.async_copy(
                cont_v.at[b],
                out_hbm.at[pl.ds(base, C), pl.ds(0, DC)], osem[b])

        start_loads(0, 0)
        run_chunk(0, 0, True, False)

        def pair_body(g, _):
            ci = 1 + g * 2
            run_chunk(ci, 1, False, False)
            run_chunk(ci + 1, 0, False, False)
            return ()

        lax.fori_loop(0, (n_chunks - 2) // 2, pair_body, (), unroll=False)
        run_chunk(n_chunks - 1, 1, False, True)
        drain_outs(1)

    return k


def kernel(past_exo_cont, past_exo_cat, tables, B, L):
    del B, L
    K, VOCAB, ED = tables.shape
    B, L, DC = past_exo_cont.shape
    N = B * L
    idx = past_exo_cat.reshape(N, K).astype(jnp.int32)
    tables_flat = tables.reshape(K * VOCAB, ED)
    cont_flat = past_exo_cont.reshape(N, DC)
    out = _make_kernel(N, K, DC, ED, VOCAB, 256)(tables_flat, idx, cont_flat)
    return out.reshape(B, L, DC + K * ED)


# speedup vs baseline: 8.8125x; 1.1069x over previous
import functools

import jax
import jax.numpy as jnp
from jax import lax
from jax.experimental import pallas as pl
from jax.experimental.pallas import tpu as pltpu
from jax.experimental.pallas import tpu_sc as plsc

NC, NS = 2, 16
NW = NC * NS
LANES = 16


def _make_kernel(B, L, DC, K, ED, VC):
    D_OUT = DC + K * ED
    BC = B // 128
    DH = D_OUT // 8
    CH = DC // 8
    EPT = K * ED // NW
    mesh = plsc.VectorSubcoreMesh(core_axis_name="c", subcore_axis_name="s")

    @functools.partial(
        pl.kernel,
        out_type=jax.ShapeDtypeStruct((L, DH, BC, 8, 128), jnp.float32),
        mesh=mesh,
        compiler_params=pltpu.CompilerParams(
            use_tc_tiling_on_sc=False, needs_layout_passes=False),
        scratch_types=[
            pltpu.VMEM((VC, 128), jnp.float32),
            pltpu.VMEM((2, BC, 128), jnp.int32),
            pltpu.VMEM((2, BC, 128), jnp.float32),
            pltpu.SemaphoreType.DMA,
            pltpu.SemaphoreType.DMA,
            pltpu.SemaphoreType.DMA,
            pltpu.SemaphoreType.DMA,
            pltpu.SemaphoreType.DMA,
            pltpu.SemaphoreType.DMA,
        ],
    )
    def k(tab_hbm, cat_hbm, cont_hbm, out_hbm,
          trow_v, idx_v, orow_v, tsem, isem0, isem1, osem0, osem1, csem):
        isem = (isem0, isem1)
        osem = (osem0, osem1)
        wid = lax.axis_index("s") * NC + lax.axis_index("c")
        kk = wid // (NW // K)
        q = wid % (NW // K)

        def fire_cont(l):
            pltpu.async_copy(cont_hbm.at[l], out_hbm.at[l, pl.ds(0, CH)], csem)

        def start_idx(l, b):
            pltpu.async_copy(cat_hbm.at[l, :, kk], idx_v.at[b], isem[b])

        def wait_idx(b):
            pltpu.make_async_copy(
                cat_hbm.at[0, :, 0], idx_v.at[b], isem[b]).wait()

        for j in range(EPT):
            e = q * EPT + j
            eh_t, el_t = e // 8, e % 8
            d = DC + kk * ED + e
            dh, dl = d // 8, d % 8

            cp_t = pltpu.async_copy(
                tab_hbm.at[kk, eh_t, :, el_t], trow_v, tsem)
            start_idx(0, 0)
            cp_t.wait()

            def l_body(l, _):
                b = l % 2

                @pl.when(b == 0)
                def _():
                    wait_idx(0)

                @pl.when(b == 1)
                def _():
                    wait_idx(1)

                @pl.when(l < L - 1)
                def _():
                    @pl.when(b == 0)
                    def _():
                        start_idx(l + 1, 1)

                    @pl.when(b == 1)
                    def _():
                        start_idx(l + 1, 0)

                @pl.when(l >= 2)
                def _():
                    @pl.when(b == 0)
                    def _():
                        pltpu.make_async_copy(
                            orow_v.at[0],
                            out_hbm.at[0, dh, :, dl], osem[0]).wait()

                    @pl.when(b == 1)
                    def _():
                        pltpu.make_async_copy(
                            orow_v.at[1],
                            out_hbm.at[0, dh, :, dl], osem[1]).wait()

                def compute(b_):
                    def bc_body(c, _):
                        for s in range(8):
                            v = idx_v[b_, c, pl.ds(s * LANES, LANES)]
                            hi = lax.shift_right_logical(v, 7)
                            lo = lax.bitwise_and(v, 127)
                            g = plsc.load_gather(trow_v, [hi, lo])
                            orow_v[b_, c, pl.ds(s * LANES, LANES)] = g
                        return ()
                    lax.fori_loop(0, BC, bc_body, (), unroll=False)

                @pl.when(b == 0)
                def _():
                    compute(0)
                    pltpu.async_copy(
                        orow_v.at[0], out_hbm.at[l, dh, :, dl], osem[0])

                @pl.when(b == 1)
                def _():
                    compute(1)
                    pltpu.async_copy(
                        orow_v.at[1], out_hbm.at[l, dh, :, dl], osem[1])
                return ()

            lax.fori_loop(0, L, l_body, (), unroll=False)
            pltpu.make_async_copy(
                orow_v.at[0], out_hbm.at[0, dh, :, dl], osem[0]).wait()
            pltpu.make_async_copy(
                orow_v.at[1], out_hbm.at[0, dh, :, dl], osem[1]).wait()

        n_my_cont = (L - 1 - wid) // NW + 1
        def cont_body(i, _):
            fire_cont(wid + i * NW)
            return ()
        lax.fori_loop(0, n_my_cont, cont_body, (), unroll=False)

        def cont_drain(i, _):
            pltpu.make_async_copy(
                cont_hbm.at[0], out_hbm.at[0, pl.ds(0, CH)], csem).wait()
            return ()
        lax.fori_loop(0, n_my_cont, cont_drain, (), unroll=False)

    return k


def kernel(past_exo_cont, past_exo_cat, tables, B, L):
    del B, L
    K, VOCAB, ED = tables.shape
    B, L, DC = past_exo_cont.shape
    VPAD = -VOCAB % 128
    VC = (VOCAB + VPAD) // 128
    cat4 = past_exo_cat.astype(jnp.int32).reshape(
        B // 128, 128, L, K).transpose(2, 0, 3, 1)
    cont5 = past_exo_cont.reshape(
        B // 128, 128, L, DC // 8, 8).transpose(2, 3, 0, 4, 1)
    tab5 = jnp.pad(tables, ((0, 0), (0, VPAD), (0, 0))).reshape(
        K, VC, 128, ED // 8, 8).transpose(0, 3, 1, 4, 2)
    out5 = _make_kernel(B, L, DC, K, ED, VC)(tab5, cat4, cont5)
    out = out5.transpose(2, 4, 0, 1, 3).reshape(B, L, DC + K * ED)
    return out


# speedup vs baseline: 13.7795x; 1.5636x over previous
import functools

import jax
import jax.numpy as jnp
from jax import lax
from jax.experimental import pallas as pl
from jax.experimental.pallas import tpu as pltpu
from jax.experimental.pallas import tpu_sc as plsc

NC, NS = 2, 16
NW = NC * NS
LANES = 16


def _make_kernel(B, L, DC, K, ED, VC):
    D_OUT = DC + K * ED
    BC = B // 128
    DH = D_OUT // 8
    CH = DC // 8
    EPT = K * ED // NW
    mesh = plsc.VectorSubcoreMesh(core_axis_name="c", subcore_axis_name="s")

    @functools.partial(
        pl.kernel,
        out_type=jax.ShapeDtypeStruct((L, DH, BC, 8, 128), jnp.float32),
        mesh=mesh,
        compiler_params=pltpu.CompilerParams(
            use_tc_tiling_on_sc=False, needs_layout_passes=False),
        scratch_types=[
            pltpu.VMEM((VC, 128), jnp.float32),
            pltpu.VMEM((2, BC, 128), jnp.int32),
            pltpu.VMEM((2, BC, 128), jnp.float32),
            pltpu.SemaphoreType.DMA,
            pltpu.SemaphoreType.DMA,
            pltpu.SemaphoreType.DMA,
            pltpu.SemaphoreType.DMA,
            pltpu.SemaphoreType.DMA,
            pltpu.SemaphoreType.DMA,
        ],
    )
    def k(tab_hbm, cat_hbm, cont_hbm, out_hbm,
          trow_v, idx_v, orow_v, tsem, isem0, isem1, osem0, osem1, csem):
        isem = (isem0, isem1)
        osem = (osem0, osem1)
        wid = lax.axis_index("s") * NC + lax.axis_index("c")
        kk = wid // (NW // K)
        q = wid % (NW // K)

        def fire_cont(l):
            pltpu.async_copy(cont_hbm.at[l], out_hbm.at[l, pl.ds(0, CH)], csem)

        def start_idx(l, b):
            pltpu.async_copy(cat_hbm.at[l, :, kk], idx_v.at[b], isem[b])

        def wait_idx(b):
            pltpu.make_async_copy(
                cat_hbm.at[0, :, 0], idx_v.at[b], isem[b]).wait()

        n_my_cont = (L - 1 - wid) // NW + 1
        def cont_body(i, _):
            fire_cont(wid + i * NW)
            return ()
        lax.fori_loop(0, n_my_cont, cont_body, (), unroll=False)

        for j in range(EPT):
            e = q * EPT + j
            eh_t, el_t = e // 8, e % 8
            d = DC + kk * ED + e
            dh, dl = d // 8, d % 8

            cp_t = pltpu.async_copy(
                tab_hbm.at[kk, eh_t, :, el_t], trow_v, tsem)
            start_idx(0, 0)
            cp_t.wait()

            def compute(b_):
                def bc_body(c, _):
                    vs = [idx_v[b_, c, pl.ds(s * LANES, LANES)]
                          for s in range(8)]
                    his = [lax.shift_right_logical(v, 7) for v in vs]
                    los = [lax.bitwise_and(v, 127) for v in vs]
                    gs = [plsc.load_gather(trow_v, [hi, lo])
                          for hi, lo in zip(his, los)]
                    for s, g in enumerate(gs):
                        orow_v[b_, c, pl.ds(s * LANES, LANES)] = g
                    return ()
                lax.fori_loop(0, BC, bc_body, (), unroll=False)

            def pair_body(h, _):
                for b in (0, 1):
                    l = 2 * h + b
                    wait_idx(b)

                    @pl.when(l < L - 1)
                    def _():
                        start_idx(l + 1, 1 - b)

                    @pl.when(l >= 2)
                    def _():
                        pltpu.make_async_copy(
                            orow_v.at[b],
                            out_hbm.at[0, dh, :, dl], osem[b]).wait()

                    compute(b)
                    pltpu.async_copy(
                        orow_v.at[b], out_hbm.at[l, dh, :, dl], osem[b])
                return ()

            lax.fori_loop(0, L // 2, pair_body, (), unroll=False)
            pltpu.make_async_copy(
                orow_v.at[0], out_hbm.at[0, dh, :, dl], osem[0]).wait()
            pltpu.make_async_copy(
                orow_v.at[1], out_hbm.at[0, dh, :, dl], osem[1]).wait()

        def cont_drain(i, _):
            pltpu.make_async_copy(
                cont_hbm.at[0], out_hbm.at[0, pl.ds(0, CH)], csem).wait()
            return ()
        lax.fori_loop(0, n_my_cont, cont_drain, (), unroll=False)

    return k


def kernel(past_exo_cont, past_exo_cat, tables, B, L):
    del B, L
    K, VOCAB, ED = tables.shape
    B, L, DC = past_exo_cont.shape
    VPAD = -VOCAB % 128
    VC = (VOCAB + VPAD) // 128
    cat4 = past_exo_cat.astype(jnp.int32).reshape(
        B // 128, 128, L, K).transpose(2, 0, 3, 1)
    cont5 = past_exo_cont.reshape(
        B // 128, 128, L, DC // 8, 8).transpose(2, 3, 0, 4, 1)
    tab5 = jnp.pad(tables, ((0, 0), (0, VPAD), (0, 0))).reshape(
        K, VC, 128, ED // 8, 8).transpose(0, 3, 1, 4, 2)
    out5 = _make_kernel(B, L, DC, K, ED, VC)(tab5, cat4, cont5)
    out = out5.transpose(2, 4, 0, 1, 3).reshape(B, L, DC + K * ED)
    return out


# speedup vs baseline: 13.8006x; 1.0015x over previous
import functools

import jax
import jax.numpy as jnp
from jax import lax
from jax.experimental import pallas as pl
from jax.experimental.pallas import tpu as pltpu
from jax.experimental.pallas import tpu_sc as plsc

NC, NS = 2, 16
NW = NC * NS
LANES = 16


def _make_kernel(B, L, DC, K, ED, VC):
    D_OUT = DC + K * ED
    BC = B // 128
    DH = D_OUT // 8
    CH = DC // 8
    EPT = K * ED // NW
    mesh = plsc.VectorSubcoreMesh(core_axis_name="c", subcore_axis_name="s")

    @functools.partial(
        pl.kernel,
        out_type=jax.ShapeDtypeStruct((L, DH, BC, 8, 128), jnp.float32),
        mesh=mesh,
        compiler_params=pltpu.CompilerParams(
            use_tc_tiling_on_sc=False, needs_layout_passes=False),
        scratch_types=[
            pltpu.VMEM((VC, 128), jnp.float32),
            pltpu.VMEM((4, BC, 128), jnp.int32),
            pltpu.VMEM((2, BC, 128), jnp.float32),
            pltpu.SemaphoreType.DMA,
            pltpu.SemaphoreType.DMA,
            pltpu.SemaphoreType.DMA,
            pltpu.SemaphoreType.DMA,
            pltpu.SemaphoreType.DMA,
            pltpu.SemaphoreType.DMA,
            pltpu.SemaphoreType.DMA,
            pltpu.SemaphoreType.DMA,
        ],
    )
    def k(tab_hbm, cat_hbm, cont_hbm, out_hbm,
          trow_v, idx_v, orow_v, tsem,
          isem0, isem1, isem2, isem3, osem0, osem1, csem):
        isem = (isem0, isem1, isem2, isem3)
        osem = (osem0, osem1)
        wid = lax.axis_index("c") * NS + lax.axis_index("s")
        kk = wid // (NW // K)
        q = wid % (NW // K)

        def fire_cont(l):
            pltpu.async_copy(cont_hbm.at[l], out_hbm.at[l, pl.ds(0, CH)], csem)

        def start_idx(l, b):
            pltpu.async_copy(cat_hbm.at[l, :, kk], idx_v.at[b], isem[b])

        def wait_idx(b):
            pltpu.make_async_copy(
                cat_hbm.at[0, :, 0], idx_v.at[b], isem[b]).wait()

        n_my_cont = (L - 1 - wid) // NW + 1
        def cont_body(i, _):
            fire_cont(wid + i * NW)
            return ()
        lax.fori_loop(0, n_my_cont, cont_body, (), unroll=False)

        for j in range(EPT):
            e = q * EPT + j
            eh_t, el_t = e // 8, e % 8
            d = DC + kk * ED + e
            dh, dl = d // 8, d % 8

            cp_t = pltpu.async_copy(
                tab_hbm.at[kk, eh_t, :, el_t], trow_v, tsem)
            for p in range(4):
                start_idx(p, p)
            cp_t.wait()

            def compute_q(b_, o_):
                def bc_body(c, _):
                    vs = [idx_v[b_, c, pl.ds(s * LANES, LANES)]
                          for s in range(8)]
                    his = [lax.shift_right_logical(v, 7) for v in vs]
                    los = [lax.bitwise_and(v, 127) for v in vs]
                    gs = [plsc.load_gather(trow_v, [hi, lo])
                          for hi, lo in zip(his, los)]
                    for s, g in enumerate(gs):
                        orow_v[o_, c, pl.ds(s * LANES, LANES)] = g
                    return ()
                lax.fori_loop(0, BC, bc_body, (), unroll=False)

            def quad_body(h, _):
                for p in range(4):
                    l = 4 * h + p
                    o = p % 2
                    wait_idx(p)

                    @pl.when(l >= 2)
                    def _():
                        pltpu.make_async_copy(
                            orow_v.at[o],
                            out_hbm.at[0, dh, :, dl], osem[o]).wait()

                    compute_q(p, o)
                    pltpu.async_copy(
                        orow_v.at[o], out_hbm.at[l, dh, :, dl], osem[o])

                    @pl.when(l + 4 < L)
                    def _():
                        start_idx(l + 4, p)
                return ()

            lax.fori_loop(0, L // 4, quad_body, (), unroll=False)
            pltpu.make_async_copy(
                orow_v.at[0], out_hbm.at[0, dh, :, dl], osem[0]).wait()
            pltpu.make_async_copy(
                orow_v.at[1], out_hbm.at[0, dh, :, dl], osem[1]).wait()

        def cont_drain(i, _):
            pltpu.make_async_copy(
                cont_hbm.at[0], out_hbm.at[0, pl.ds(0, CH)], csem).wait()
            return ()
        lax.fori_loop(0, n_my_cont, cont_drain, (), unroll=False)

    return k


def kernel(past_exo_cont, past_exo_cat, tables, B, L):
    del B, L
    K, VOCAB, ED = tables.shape
    B, L, DC = past_exo_cont.shape
    VPAD = -VOCAB % 128
    VC = (VOCAB + VPAD) // 128
    cat4 = past_exo_cat.astype(jnp.int32).reshape(
        B // 128, 128, L, K).transpose(2, 0, 3, 1)
    cont5 = past_exo_cont.reshape(
        B // 128, 128, L, DC // 8, 8).transpose(2, 3, 0, 4, 1)
    tab5 = jnp.pad(tables, ((0, 0), (0, VPAD), (0, 0))).reshape(
        K, VC, 128, ED // 8, 8).transpose(0, 3, 1, 4, 2)
    out5 = _make_kernel(B, L, DC, K, ED, VC)(tab5, cat4, cont5)
    out = out5.transpose(2, 4, 0, 1, 3).reshape(B, L, DC + K * ED)
    return out
